# baseline (speedup 1.0000x reference)
import functools

import jax
import jax.numpy as jnp
from jax import lax
from jax.experimental import pallas as pl
from jax.experimental.pallas import tpu as pltpu
from jax.experimental.pallas import tpu_sc as plsc

BATCH = 16
LANES = 16
NC = 2
NS = 16
NW = NC * NS
CHUNK = 128


def _sc_partial_sums(text3, emb_weight, n_chunks):
    d = emb_weight.shape[1]
    mesh = plsc.VectorSubcoreMesh(
        core_axis_name="c", subcore_axis_name="s",
        num_cores=NC, num_subcores=NS)

    @functools.partial(
        pl.kernel,
        out_type=jax.ShapeDtypeStruct((NC, NS, d), jnp.float32),
        mesh=mesh,
        scratch_types=[
            pltpu.VMEM((n_chunks, CHUNK), jnp.int32),
            pltpu.VMEM((CHUNK, d), jnp.float32),
            pltpu.VMEM((d,), jnp.float32),
            pltpu.SemaphoreType.DMA,
        ],
        compiler_params=pltpu.CompilerParams(use_tc_tiling_on_sc=False),
    )
    def body(text_hbm, table_hbm, out_hbm, idx_v, rows_v, acc_v, sem):
        ci = lax.axis_index("c")
        si = lax.axis_index("s")
        wid = si * NC + ci
        pltpu.sync_copy(text_hbm.at[wid], idx_v)

        zero = jnp.zeros((LANES,), jnp.float32)

        def chunk_body(j, accs):
            pltpu.async_copy(table_hbm.at[idx_v.at[j]], rows_v, sem).wait()

            def row_body(r, accs):
                a0, a1, a2, a3 = accs
                a0 = a0 + rows_v[r, pl.ds(0 * LANES, LANES)]
                a1 = a1 + rows_v[r, pl.ds(1 * LANES, LANES)]
                a2 = a2 + rows_v[r, pl.ds(2 * LANES, LANES)]
                a3 = a3 + rows_v[r, pl.ds(3 * LANES, LANES)]
                return (a0, a1, a2, a3)

            return lax.fori_loop(0, CHUNK, row_body, accs)

        accs = lax.fori_loop(0, n_chunks, chunk_body, (zero, zero, zero, zero))
        acc_v[pl.ds(0 * LANES, LANES)] = accs[0]
        acc_v[pl.ds(1 * LANES, LANES)] = accs[1]
        acc_v[pl.ds(2 * LANES, LANES)] = accs[2]
        acc_v[pl.ds(3 * LANES, LANES)] = accs[3]
        pltpu.sync_copy(acc_v, out_hbm.at[ci, si])

    return body(text3, emb_weight)


def _tc_head(partials, fcw_t, bias2d, inv_count):
    num_class = fcw_t.shape[1]

    def body(p_ref, w_ref, b_ref, o_ref):
        pooled = (p_ref[0] + p_ref[1]) * inv_count
        o_ref[...] = jnp.dot(pooled, w_ref[...],
                             preferred_element_type=jnp.float32) + b_ref[...]

    return pl.pallas_call(
        body,
        out_shape=jax.ShapeDtypeStruct((BATCH, num_class), jnp.float32),
    )(partials, fcw_t, bias2d)


def kernel(text, emb_weight, fc_weight, fc_bias):
    n = text.shape[0]
    count = n // BATCH
    per_w = n // NW
    n_chunks = per_w // CHUNK
    assert n % (NW * CHUNK) == 0 and count % per_w == 0

    text3 = text.astype(jnp.int32).reshape(NW, n_chunks, CHUNK)
    partials = _sc_partial_sums(text3, emb_weight, n_chunks)
    fcw_t = fc_weight.T
    bias2d = fc_bias.reshape(1, -1)
    return _tc_head(partials, fcw_t, bias2d, 1.0 / count)


# speedup vs baseline: 1.0944x; 1.0944x over previous
import functools

import jax
import jax.numpy as jnp
from jax import lax
from jax.experimental import pallas as pl
from jax.experimental.pallas import tpu as pltpu
from jax.experimental.pallas import tpu_sc as plsc

BATCH = 16
LANES = 16
NC = 2
NS = 16
NW = NC * NS
CHUNK = 128
NBUF = 4
RU = 8


def _sc_partial_sums(text3, emb_weight, n_chunks):
    d = emb_weight.shape[1]
    mesh = plsc.VectorSubcoreMesh(
        core_axis_name="c", subcore_axis_name="s",
        num_cores=NC, num_subcores=NS)

    @functools.partial(
        pl.kernel,
        out_type=jax.ShapeDtypeStruct((NC, NS, d), jnp.float32),
        mesh=mesh,
        scratch_types=[
            pltpu.VMEM((n_chunks, CHUNK), jnp.int32),
            pltpu.VMEM((NBUF, CHUNK, d), jnp.float32),
            pltpu.VMEM((d,), jnp.float32),
            pltpu.SemaphoreType.DMA,
            pltpu.SemaphoreType.DMA,
            pltpu.SemaphoreType.DMA,
            pltpu.SemaphoreType.DMA,
        ],
        compiler_params=pltpu.CompilerParams(use_tc_tiling_on_sc=False),
    )
    def body(text_hbm, table_hbm, out_hbm, idx_v, rows_v, acc_v, *sems):
        ci = lax.axis_index("c")
        si = lax.axis_index("s")
        wid = si * NC + ci
        pltpu.sync_copy(text_hbm.at[wid], idx_v)

        for p in range(NBUF):
            pltpu.async_copy(table_hbm.at[idx_v.at[p]], rows_v.at[p], sems[p])

        zero = jnp.zeros((LANES,), jnp.float32)

        def accum_chunk(rows_p, accs):
            def row_body(r, accs8):
                accs8 = list(accs8)
                for u in range(0, RU, 2):
                    row = r * RU + u
                    for k in range(4):
                        accs8[k] = accs8[k] + rows_p[
                            row, pl.ds(k * LANES, LANES)]
                        accs8[4 + k] = accs8[4 + k] + rows_p[
                            row + 1, pl.ds(k * LANES, LANES)]
                return tuple(accs8)

            accs8 = accs + (zero, zero, zero, zero)
            accs8 = lax.fori_loop(0, CHUNK // RU, row_body, accs8)
            return tuple(accs8[k] + accs8[4 + k] for k in range(4))

        def outer(t, accs):
            for p in range(NBUF):
                j = t * NBUF + p
                pltpu.make_async_copy(
                    table_hbm.at[idx_v.at[j]], rows_v.at[p], sems[p]).wait()
                accs = accum_chunk(rows_v.at[p], accs)

                @pl.when(j + NBUF < n_chunks)
                def _():
                    pltpu.async_copy(
                        table_hbm.at[idx_v.at[j + NBUF]], rows_v.at[p], sems[p])
            return accs

        accs = lax.fori_loop(0, n_chunks // NBUF, outer,
                             (zero, zero, zero, zero))
        acc_v[pl.ds(0 * LANES, LANES)] = accs[0]
        acc_v[pl.ds(1 * LANES, LANES)] = accs[1]
        acc_v[pl.ds(2 * LANES, LANES)] = accs[2]
        acc_v[pl.ds(3 * LANES, LANES)] = accs[3]
        pltpu.sync_copy(acc_v, out_hbm.at[ci, si])

    return body(text3, emb_weight)


def _tc_head(partials, fcw_t, bias2d, inv_count):
    num_class = fcw_t.shape[1]

    def body(p_ref, w_ref, b_ref, o_ref):
        pooled = (p_ref[0] + p_ref[1]) * inv_count
        o_ref[...] = jnp.dot(pooled, w_ref[...],
                             preferred_element_type=jnp.float32) + b_ref[...]

    return pl.pallas_call(
        body,
        out_shape=jax.ShapeDtypeStruct((BATCH, num_class), jnp.float32),
    )(partials, fcw_t, bias2d)


def kernel(text, emb_weight, fc_weight, fc_bias):
    n = text.shape[0]
    count = n // BATCH
    per_w = n // NW
    n_chunks = per_w // CHUNK
    assert n % (NW * CHUNK) == 0 and count % per_w == 0
    assert (per_w // CHUNK) % NBUF == 0

    text3 = text.astype(jnp.int32).reshape(NW, n_chunks, CHUNK)
    partials = _sc_partial_sums(text3, emb_weight, n_chunks)
    fcw_t = fc_weight.T
    bias2d = fc_bias.reshape(1, -1)
    return _tc_head(partials, fcw_t, bias2d, 1.0 / count)
